# baseline (speedup 1.0000x reference)
import functools

import jax
import jax.numpy as jnp
from jax import lax
from jax.experimental import pallas as pl
from jax.experimental.pallas import tpu as pltpu
from jax.experimental.pallas import tpu_sc as plsc

_LANES = 128
_NB = 8
_SR = 14
_NSUB = 16
_NCORE = 2

_NL = 102400
_NC = 51200
_ND = 25600


def _relu(x):
    return jnp.maximum(x, 0.0)


def _dot(x, w):
    return jnp.dot(x, w, preferred_element_type=jnp.float32)


def _stack16(ref, f):
    x = ref[...]
    return jnp.concatenate(
        [x[:, t * f:(t + 1) * f] for t in range(16)], axis=0)


def _fuse16(y, b, q, f):
    return jnp.concatenate(
        [y[t * b:(t + 1) * b, q * f:(q + 1) * f] for t in range(16)], axis=1)


def _lit_encoder(lab, w, b):
    m = _NL // 16
    blk = 320
    labv = lab.reshape(16, m, 8)

    def body(l_ref, w_ref, b_ref, o_ref):
        x = jnp.concatenate([l_ref[t] for t in range(16)], axis=0)
        y = _relu(_dot(x, w_ref[...]) + b_ref[...])
        for q in range(4):
            o_ref[q] = _fuse16(y, blk, q, 16)

    out = pl.pallas_call(
        body,
        grid=(m // blk,),
        in_specs=[
            pl.BlockSpec((16, blk, 8), lambda i: (0, i, 0)),
            pl.BlockSpec((8, 64), lambda i: (0, 0)),
            pl.BlockSpec((1, 64), lambda i: (0, 0)),
        ],
        out_specs=pl.BlockSpec((4, blk, 256), lambda i: (0, i, 0)),
        out_shape=jax.ShapeDtypeStruct((4, m, 256), jnp.float32),
    )(labv, w, b.reshape(1, 64))
    return out.reshape(4, _NL, 16)


def _clause_encoder(m_parts, lab, w, b):
    m = _NC // 16
    blk = 160
    mv = m_parts.reshape(2, m, 512)
    labv = lab.reshape(16, m, 8)

    def body(m_ref, l_ref, w_ref, b_ref, o_ref):
        xs = [_stack16(m_ref[p], 32) for p in range(2)]
        xs.append(jnp.concatenate([l_ref[t] for t in range(16)], axis=0))
        x = jnp.concatenate(xs, axis=1)
        y = _relu(_dot(x, w_ref[...]) + b_ref[...])
        for q in range(8):
            o_ref[q] = _fuse16(y, blk, q, 8)

    out = pl.pallas_call(
        body,
        grid=(m // blk,),
        in_specs=[
            pl.BlockSpec((2, blk, 512), lambda i: (0, i, 0)),
            pl.BlockSpec((16, blk, 8), lambda i: (0, i, 0)),
            pl.BlockSpec((72, 64), lambda i: (0, 0)),
            pl.BlockSpec((1, 64), lambda i: (0, 0)),
        ],
        out_specs=pl.BlockSpec((8, blk, 128), lambda i: (0, i, 0)),
        out_shape=jax.ShapeDtypeStruct((8, m, 128), jnp.float32),
    )(mv, labv, w, b.reshape(1, 64))
    return out.reshape(8, _NC, 8)


def _lit_encoder2(m_parts, lab, w, b):
    m = _NL // 16
    blk = 320
    mv = m_parts.reshape(2, m, 512)
    labv = lab.reshape(16, m, 8)

    def body(m_ref, l_ref, w_ref, b_ref, o_ref):
        xs = [_stack16(m_ref[p], 32) for p in range(2)]
        xs.append(jnp.concatenate([l_ref[t] for t in range(16)], axis=0))
        x = jnp.concatenate(xs, axis=1)
        y = _relu(_dot(x, w_ref[...]) + b_ref[...])
        for q in range(4):
            o_ref[q] = _fuse16(y, blk, q, 16)

    out = pl.pallas_call(
        body,
        grid=(m // blk,),
        in_specs=[
            pl.BlockSpec((2, blk, 512), lambda i: (0, i, 0)),
            pl.BlockSpec((16, blk, 8), lambda i: (0, i, 0)),
            pl.BlockSpec((72, 64), lambda i: (0, 0)),
            pl.BlockSpec((1, 64), lambda i: (0, 0)),
        ],
        out_specs=pl.BlockSpec((4, blk, 256), lambda i: (0, i, 0)),
        out_shape=jax.ShapeDtypeStruct((4, m, 256), jnp.float32),
    )(mv, labv, w, b.reshape(1, 64))
    return out.reshape(4, _NL, 16)


def _decoder(m_parts, lab, gss, w_c2, b_c2, w_dec, b_dec):
    m = _ND // 8
    blk = 160
    mv = m_parts.reshape(2, m, 256)
    labv = lab.reshape(8, m, 16)
    gssv = gss.reshape(8, m, 256)

    def body(m_ref, c_ref, g_ref, wc_ref, bc_ref, wd_ref, bd_ref, o_ref):
        clab = jnp.concatenate([c_ref[t][:, 8:] for t in range(8)], axis=0)
        xs = [jnp.concatenate([m_ref[p][:, 32 * t:32 * (t + 1)]
                               for t in range(8)], axis=0) for p in range(2)]
        x = jnp.concatenate(xs + [clab], axis=1)
        h = _relu(_dot(x, wc_ref[...]) + bc_ref[...])
        g = jnp.concatenate([g_ref[t][:, 128:] for t in range(8)], axis=0)
        z = _dot(jnp.concatenate([g, h, clab], axis=1), wd_ref[...])
        z = z + bd_ref[...]
        for t in range(8):
            o_ref[t] = z[t * blk:(t + 1) * blk]

    out = pl.pallas_call(
        body,
        grid=(m // blk,),
        in_specs=[
            pl.BlockSpec((2, blk, 256), lambda i: (0, i, 0)),
            pl.BlockSpec((8, blk, 16), lambda i: (0, i, 0)),
            pl.BlockSpec((8, blk, 256), lambda i: (0, i, 0)),
            pl.BlockSpec((72, 64), lambda i: (0, 0)),
            pl.BlockSpec((1, 64), lambda i: (0, 0)),
            pl.BlockSpec((200, 2), lambda i: (0, 0)),
            pl.BlockSpec((1, 2), lambda i: (0, 0)),
        ],
        out_specs=pl.BlockSpec((8, blk, 2), lambda i: (0, i, 0)),
        out_shape=jax.ShapeDtypeStruct((8, m, 2), jnp.float32),
    )(mv, labv, gssv, w_c2, b_c2.reshape(1, 64), w_dec, b_dec.reshape(1, 2))
    return out.reshape(_ND, 2)


@functools.lru_cache(maxsize=None)
def _make_seg_kernel(nparts, n_src, n_dst, fdim, rows, passes_per_core,
                     pack=1):
    n_dst_pad = n_dst + 128
    zrows = n_dst_pad // _NSUB
    drows = n_dst // _NSUB
    rows_per_tile = rows // _NSUB

    mesh = plsc.VectorSubcoreMesh(core_axis_name="c", subcore_axis_name="s",
                                  num_cores=_NCORE, num_subcores=_NSUB)

    def body(table_h, src_h, dst_h, zeros_h, out_h,
             li_v, di_v, li2_v, di2_v, rows_v, acc_sh, isem, gsem, ssem):
        c = lax.axis_index("c")
        s = lax.axis_index("s")
        for r in range(passes_per_core):
            q = c * passes_per_core + r
            dummy = table_h.at[q].at[pl.ds(0, _LANES)]

            def drain(sem, b):
                pltpu.make_async_copy(dummy, rows_v.at[b], sem.at[b]).wait()

            pltpu.sync_copy(zeros_h, acc_sh.at[pl.ds(s * zrows, zrows)])
            plsc.subcore_barrier()

            def half(k, first, li, di):
                base = s * rows_per_tile + k * _SR
                ci1 = pltpu.async_copy(src_h.at[pl.ds(base, _SR)], li, isem)
                ci2 = pltpu.async_copy(dst_h.at[pl.ds(base, _SR)], di, isem)
                ci1.wait()
                ci2.wait()
                for b in range(_SR):
                    if first is None:
                        drain(ssem, b)
                    else:
                        @pl.when(first == 0)
                        def _():
                            drain(ssem, b)
                    pltpu.async_copy(table_h.at[q].at[li.at[b]],
                                     rows_v.at[b], gsem.at[b])
                for b in range(_SR):
                    drain(gsem, b)
                    pltpu.async_copy(rows_v.at[b], acc_sh.at[di.at[b]],
                                     ssem.at[b], add=True)

            def chunk2(i, carry):
                half(2 * i, jnp.where(i > 0, 0, 1), li_v, di_v)
                half(2 * i + 1, None, li2_v, di2_v)
                return carry

            lax.fori_loop(0, rows_per_tile // (2 * _SR), chunk2, 0)
            for b in range(_SR):
                drain(ssem, b)
            plsc.subcore_barrier()
            pltpu.sync_copy(
                acc_sh.at[pl.ds(s * drows, drows)],
                out_h.at[q // pack].at[pl.ds(s * drows, drows),
                                       pl.ds(fdim * (q % pack), fdim)])
            if r + 1 < passes_per_core:
                plsc.subcore_barrier()

    return pl.kernel(
        body,
        out_type=jax.ShapeDtypeStruct((nparts // pack, n_dst, fdim * pack),
                                      jnp.float32),
        mesh=mesh,
        scratch_types=[
            pltpu.VMEM((_SR, _LANES), jnp.int32),
            pltpu.VMEM((_SR, _LANES), jnp.int32),
            pltpu.VMEM((_SR, _LANES), jnp.int32),
            pltpu.VMEM((_SR, _LANES), jnp.int32),
            pltpu.VMEM((_SR, _LANES, fdim), jnp.float32),
            pltpu.VMEM_SHARED((n_dst_pad, fdim), jnp.float32),
            pltpu.SemaphoreType.DMA,
            pltpu.SemaphoreType.DMA((_SR,)),
            pltpu.SemaphoreType.DMA((_SR,)),
        ],
        compiler_params=pltpu.CompilerParams(use_tc_tiling_on_sc=False),
    )


def _pad_rows(x, n):
    return jnp.concatenate(
        [x, jnp.zeros((n - x.shape[0],) + x.shape[1:], x.dtype)])


def _pad_edges(idx, rows, fill):
    pad = rows * _LANES - idx.shape[0]
    return jnp.concatenate([idx, jnp.full((pad,), fill, jnp.int32)]).reshape(
        rows, _LANES)


def kernel(gss, lit_labels, clause_labels, edge_lit, edge_cl,
           W_lit, b_lit, W_c, b_c, W_l2, b_l2, W_c2, b_c2, W_dec, b_dec):
    e = edge_lit.shape[0]
    unit = _LANES * _NSUB * _NB
    rows = -(-e // unit) * _NSUB * _NB

    ll = _pad_rows(lit_labels, _NL)
    cl = _pad_rows(clause_labels, _NC)
    gssp = _pad_rows(gss, _NC)

    pl16_lit = 16 * (edge_lit % (_NL // 16)) + edge_lit // (_NL // 16)
    pl16_cl = 16 * (edge_cl % (_NC // 16)) + edge_cl // (_NC // 16)
    lrn_half = (edge_cl - 1) // 2
    pl8_d = jnp.where(edge_cl % 2 == 1,
                      8 * (lrn_half % (_ND // 8)) + lrn_half // (_ND // 8),
                      _ND)

    src_l = _pad_edges(pl16_lit, rows, 0)
    dst_c = _pad_edges(pl16_cl, rows, _NC)
    src_c = _pad_edges(pl16_cl, rows, 0)
    dst_l = _pad_edges(pl16_lit, rows, _NL)
    dst_d = _pad_edges(pl8_d, rows, _ND)

    z_c = jnp.zeros(((_NC + 128) // _NSUB, 16), jnp.float32)
    z_l = jnp.zeros(((_NL + 128) // _NSUB, 8), jnp.float32)
    z_d = jnp.zeros(((_ND + 128) // _NSUB, 16), jnp.float32)

    seg1 = _make_seg_kernel(4, _NL, _NC, 16, rows, 2, pack=2)
    seg2 = _make_seg_kernel(8, _NC, _NL, 8, rows, 4, pack=4)
    seg3 = _make_seg_kernel(4, _NL, _ND, 16, rows, 2, pack=2)

    h_l0 = _lit_encoder(ll, W_lit, b_lit)
    m_c = seg1(h_l0, src_l, dst_c, z_c)
    h_c = _clause_encoder(m_c, cl, W_c, b_c)
    m_l = seg2(h_c, src_c, dst_l, z_l)
    vembs = _lit_encoder2(m_l, ll, W_l2, b_l2)
    m_d = seg3(vembs, src_l, dst_d, z_d)
    lg = _decoder(m_d, cl, gssp, W_c2, b_c2, W_dec, b_dec)
    return lg[:25000]


# speedup vs baseline: 1.4562x; 1.4562x over previous
import functools

import jax
import jax.numpy as jnp
from jax import lax
from jax.experimental import pallas as pl
from jax.experimental.pallas import tpu as pltpu
from jax.experimental.pallas import tpu_sc as plsc

_LANES = 128
_NB = 8
_SR = 14
_NSUB = 16
_NCORE = 2

_NL = 102400
_NC = 51200
_ND = 25600


def _relu(x):
    return jnp.maximum(x, 0.0)


def _dot(x, w):
    return jnp.dot(x, w, preferred_element_type=jnp.float32)


def _stack16(ref, f):
    x = ref[...]
    return jnp.concatenate(
        [x[:, t * f:(t + 1) * f] for t in range(16)], axis=0)


def _fuse16(y, b, q, f):
    return jnp.concatenate(
        [y[t * b:(t + 1) * b, q * f:(q + 1) * f] for t in range(16)], axis=1)


def _lit_encoder(lab, w, b):
    m = _NL // 16
    blk = 320
    labv = lab.reshape(16, m, 8)

    def body(l_ref, w_ref, b_ref, o_ref):
        x = jnp.concatenate([l_ref[t] for t in range(16)], axis=0)
        y = _relu(_dot(x, w_ref[...]) + b_ref[...])
        for q in range(4):
            o_ref[q] = _fuse16(y, blk, q, 16)

    out = pl.pallas_call(
        body,
        grid=(m // blk,),
        in_specs=[
            pl.BlockSpec((16, blk, 8), lambda i: (0, i, 0)),
            pl.BlockSpec((8, 64), lambda i: (0, 0)),
            pl.BlockSpec((1, 64), lambda i: (0, 0)),
        ],
        out_specs=pl.BlockSpec((4, blk, 256), lambda i: (0, i, 0)),
        out_shape=jax.ShapeDtypeStruct((4, m, 256), jnp.float32),
    )(labv, w, b.reshape(1, 64))
    return out.reshape(4, _NL, 16)


def _clause_encoder(m_parts, lab, w, b):
    m = _NC // 16
    blk = 160
    mv = m_parts.reshape(2, m, 512)
    labv = lab.reshape(16, m, 8)

    def body(m_ref, l_ref, w_ref, b_ref, o_ref):
        xs = [_stack16(m_ref[p], 32) for p in range(2)]
        xs.append(jnp.concatenate([l_ref[t] for t in range(16)], axis=0))
        x = jnp.concatenate(xs, axis=1)
        y = _relu(_dot(x, w_ref[...]) + b_ref[...])
        for q in range(8):
            o_ref[q] = _fuse16(y, blk, q, 8)

    out = pl.pallas_call(
        body,
        grid=(m // blk,),
        in_specs=[
            pl.BlockSpec((2, blk, 512), lambda i: (0, i, 0)),
            pl.BlockSpec((16, blk, 8), lambda i: (0, i, 0)),
            pl.BlockSpec((72, 64), lambda i: (0, 0)),
            pl.BlockSpec((1, 64), lambda i: (0, 0)),
        ],
        out_specs=pl.BlockSpec((8, blk, 128), lambda i: (0, i, 0)),
        out_shape=jax.ShapeDtypeStruct((8, m, 128), jnp.float32),
    )(mv, labv, w, b.reshape(1, 64))
    return out.reshape(8, _NC, 8)


def _lit_encoder2(m_parts, lab, w, b):
    m = _NL // 16
    blk = 320
    mv = m_parts.reshape(2, m, 512)
    labv = lab.reshape(16, m, 8)

    def body(m_ref, l_ref, w_ref, b_ref, o_ref):
        xs = [_stack16(m_ref[p], 32) for p in range(2)]
        xs.append(jnp.concatenate([l_ref[t] for t in range(16)], axis=0))
        x = jnp.concatenate(xs, axis=1)
        y = _relu(_dot(x, w_ref[...]) + b_ref[...])
        for q in range(4):
            o_ref[q] = _fuse16(y, blk, q, 16)

    out = pl.pallas_call(
        body,
        grid=(m // blk,),
        in_specs=[
            pl.BlockSpec((2, blk, 512), lambda i: (0, i, 0)),
            pl.BlockSpec((16, blk, 8), lambda i: (0, i, 0)),
            pl.BlockSpec((72, 64), lambda i: (0, 0)),
            pl.BlockSpec((1, 64), lambda i: (0, 0)),
        ],
        out_specs=pl.BlockSpec((4, blk, 256), lambda i: (0, i, 0)),
        out_shape=jax.ShapeDtypeStruct((4, m, 256), jnp.float32),
    )(mv, labv, w, b.reshape(1, 64))
    return out.reshape(4, _NL, 16)


def _decoder(m_parts, lab, gss, w_c2, b_c2, w_dec, b_dec):
    m = _ND // 8
    blk = 160
    mv = m_parts.reshape(2, m, 256)
    labv = lab.reshape(8, m, 16)
    gssv = gss.reshape(8, m, 256)

    def body(m_ref, c_ref, g_ref, wc_ref, bc_ref, wd_ref, bd_ref, o_ref):
        clab = jnp.concatenate([c_ref[t][:, 8:] for t in range(8)], axis=0)
        xs = [jnp.concatenate([m_ref[p][:, 32 * t:32 * (t + 1)]
                               for t in range(8)], axis=0) for p in range(2)]
        x = jnp.concatenate(xs + [clab], axis=1)
        h = _relu(_dot(x, wc_ref[...]) + bc_ref[...])
        g = jnp.concatenate([g_ref[t][:, 128:] for t in range(8)], axis=0)
        z = _dot(jnp.concatenate([g, h, clab], axis=1), wd_ref[...])
        z = z + bd_ref[...]
        for t in range(8):
            o_ref[t] = z[t * blk:(t + 1) * blk]

    out = pl.pallas_call(
        body,
        grid=(m // blk,),
        in_specs=[
            pl.BlockSpec((2, blk, 256), lambda i: (0, i, 0)),
            pl.BlockSpec((8, blk, 16), lambda i: (0, i, 0)),
            pl.BlockSpec((8, blk, 256), lambda i: (0, i, 0)),
            pl.BlockSpec((72, 64), lambda i: (0, 0)),
            pl.BlockSpec((1, 64), lambda i: (0, 0)),
            pl.BlockSpec((200, 2), lambda i: (0, 0)),
            pl.BlockSpec((1, 2), lambda i: (0, 0)),
        ],
        out_specs=pl.BlockSpec((8, blk, 2), lambda i: (0, i, 0)),
        out_shape=jax.ShapeDtypeStruct((8, m, 2), jnp.float32),
    )(mv, labv, gssv, w_c2, b_c2.reshape(1, 64), w_dec, b_dec.reshape(1, 2))
    return out.reshape(_ND, 2)


@functools.lru_cache(maxsize=None)
def _make_seg_kernel(nparts, n_src, n_dst, fdim, rows, passes_per_core,
                     pack=1):
    n_dst_pad = n_dst + 128
    zrows = n_dst_pad // _NSUB
    drows = n_dst // _NSUB
    rows_per_tile = rows // _NSUB

    mesh = plsc.VectorSubcoreMesh(core_axis_name="c", subcore_axis_name="s",
                                  num_cores=_NCORE, num_subcores=_NSUB)

    def body(table_h, src_h, dst_h, zeros_h, out_h,
             li_v, di_v, li2_v, di2_v, rows_v, acc_sh, isem, gsem, ssem):
        c = lax.axis_index("c")
        s = lax.axis_index("s")
        for r in range(passes_per_core):
            q = c * passes_per_core + r
            dummy = table_h.at[q].at[pl.ds(0, _LANES)]

            def drain(sem, b):
                pltpu.make_async_copy(dummy, rows_v.at[b], sem.at[b]).wait()

            pltpu.sync_copy(zeros_h, acc_sh.at[pl.ds(s * zrows, zrows)])
            plsc.subcore_barrier()

            def half(k, first, li, di):
                base = s * rows_per_tile + k * _SR
                ci1 = pltpu.async_copy(src_h.at[pl.ds(base, _SR)], li, isem)
                ci2 = pltpu.async_copy(dst_h.at[pl.ds(base, _SR)], di, isem)
                ci1.wait()
                ci2.wait()
                for b in range(_SR):
                    if first is None:
                        drain(ssem, b)
                    else:
                        @pl.when(first == 0)
                        def _():
                            drain(ssem, b)
                    pltpu.async_copy(table_h.at[q].at[li.at[b]],
                                     rows_v.at[b], gsem.at[b])
                for b in range(_SR):
                    drain(gsem, b)
                    pltpu.async_copy(rows_v.at[b], acc_sh.at[di.at[b]],
                                     ssem.at[b], add=True)

            def chunk2(i, carry):
                half(2 * i, jnp.where(i > 0, 0, 1), li_v, di_v)
                half(2 * i + 1, None, li2_v, di2_v)
                return carry

            lax.fori_loop(0, rows_per_tile // (2 * _SR), chunk2, 0)
            for b in range(_SR):
                drain(ssem, b)
            plsc.subcore_barrier()
            pltpu.sync_copy(
                acc_sh.at[pl.ds(s * drows, drows)],
                out_h.at[q // pack].at[pl.ds(s * drows, drows),
                                       pl.ds(fdim * (q % pack), fdim)])
            if r + 1 < passes_per_core:
                plsc.subcore_barrier()

    return pl.kernel(
        body,
        out_type=jax.ShapeDtypeStruct((nparts // pack, n_dst, fdim * pack),
                                      jnp.float32),
        mesh=mesh,
        scratch_types=[
            pltpu.VMEM((_SR, _LANES), jnp.int32),
            pltpu.VMEM((_SR, _LANES), jnp.int32),
            pltpu.VMEM((_SR, _LANES), jnp.int32),
            pltpu.VMEM((_SR, _LANES), jnp.int32),
            pltpu.VMEM((_SR, _LANES, fdim), jnp.float32),
            pltpu.VMEM_SHARED((n_dst_pad, fdim), jnp.float32),
            pltpu.SemaphoreType.DMA,
            pltpu.SemaphoreType.DMA((_SR,)),
            pltpu.SemaphoreType.DMA((_SR,)),
        ],
        compiler_params=pltpu.CompilerParams(use_tc_tiling_on_sc=False),
    )


def _pad_rows(x, n):
    return jnp.concatenate(
        [x, jnp.zeros((n - x.shape[0],) + x.shape[1:], x.dtype)])


def _pad_edges(idx, rows, fill):
    pad = rows * _LANES - idx.shape[0]
    return jnp.concatenate([idx, jnp.full((pad,), fill, jnp.int32)]).reshape(
        rows, _LANES)


def kernel(gss, lit_labels, clause_labels, edge_lit, edge_cl,
           W_lit, b_lit, W_c, b_c, W_l2, b_l2, W_c2, b_c2, W_dec, b_dec):
    e = edge_lit.shape[0]
    unit = _LANES * _NSUB * _NB
    rows = -(-e // unit) * _NSUB * _NB

    ll = _pad_rows(lit_labels, _NL)
    cl = _pad_rows(clause_labels, _NC)
    gssp = _pad_rows(gss, _NC)

    pl16_lit = 16 * (edge_lit % (_NL // 16)) + edge_lit // (_NL // 16)
    pl16_cl = 16 * (edge_cl % (_NC // 16)) + edge_cl // (_NC // 16)
    lrn_half = (edge_cl - 1) // 2
    pl8_d = jnp.where(edge_cl % 2 == 1,
                      8 * (lrn_half % (_ND // 8)) + lrn_half // (_ND // 8),
                      _ND + (lrn_half % 128))

    src_l = _pad_edges(pl16_lit, rows, 0)
    dst_c = _pad_edges(pl16_cl, rows, _NC)
    src_c = _pad_edges(pl16_cl, rows, 0)
    dst_l = _pad_edges(pl16_lit, rows, _NL)
    dst_d = _pad_edges(pl8_d, rows, _ND)

    z_c = jnp.zeros(((_NC + 128) // _NSUB, 16), jnp.float32)
    z_l = jnp.zeros(((_NL + 128) // _NSUB, 8), jnp.float32)
    z_d = jnp.zeros(((_ND + 128) // _NSUB, 16), jnp.float32)

    seg1 = _make_seg_kernel(4, _NL, _NC, 16, rows, 2, pack=2)
    seg2 = _make_seg_kernel(8, _NC, _NL, 8, rows, 4, pack=4)
    seg3 = _make_seg_kernel(4, _NL, _ND, 16, rows, 2, pack=2)

    h_l0 = _lit_encoder(ll, W_lit, b_lit)
    m_c = seg1(h_l0, src_l, dst_c, z_c)
    h_c = _clause_encoder(m_c, cl, W_c, b_c)
    m_l = seg2(h_c, src_c, dst_l, z_l)
    vembs = _lit_encoder2(m_l, ll, W_l2, b_l2)
    m_d = seg3(vembs, src_l, dst_d, z_d)
    lg = _decoder(m_d, cl, gssp, W_c2, b_c2, W_dec, b_dec)
    return lg[:25000]
